# baseline (speedup 1.0000x reference)
import functools

import jax
import jax.numpy as jnp
from jax import lax
from jax.experimental import pallas as pl
from jax.experimental.pallas import tpu as pltpu
from jax.experimental.pallas import tpu_sc as plsc

VOCAB = 100000
HID = 768
B = 4
S = 4096
N = B * S
EPS = 1e-12

NC = 2
NS = 16
NW = NC * NS
TPW = N // NW
C = 32
NCH = TPW // C
HC = HID // 16


_GDN = lax.GatherDimensionNumbers(
    offset_dims=(), collapsed_slice_dims=(0,), start_index_map=(0,))


def _lane_perm(x, perm):
    return lax.gather(x, perm[:, None], _GDN, (1,),
                      mode=lax.GatherScatterMode.PROMISE_IN_BOUNDS)


def _body(ids, tts, pids, wtab, ttab, ptab, gam, bet, out,
          idx_v, tt_v, pidx_v, wbuf, pbuf, tbuf, gv, bv, sem):
    wid = lax.axis_index("s") * NC + lax.axis_index("c")
    base = pl.multiple_of(wid * TPW, TPW)
    pltpu.sync_copy(ids.at[pl.ds(base, TPW)], idx_v)
    pltpu.sync_copy(tts.at[pl.ds(base, TPW)], tt_v)
    pltpu.sync_copy(pids.at[pl.ds(base, TPW)], pidx_v)
    pltpu.sync_copy(gam, gv)
    pltpu.sync_copy(bet, bv)

    def chunk_body(g, carry):
        off = pl.multiple_of(g * C, C)
        cp1 = pltpu.async_copy(wtab.at[idx_v.at[pl.ds(off, C)]], wbuf, sem)
        cp2 = pltpu.async_copy(ptab.at[pidx_v.at[pl.ds(off, C)]], pbuf, sem)
        cp3 = pltpu.async_copy(ttab.at[tt_v.at[pl.ds(off, C)]], tbuf, sem)
        cp1.wait()
        cp2.wait()
        cp3.wait()

        def tok_body(t, carry2):
            acc = jnp.zeros((16,), jnp.float32)
            acc2 = jnp.zeros((16,), jnp.float32)
            for j in range(HC):
                sl = pl.ds(j * 16, 16)
                x = wbuf[t, sl] + pbuf[t, sl] + tbuf[t, sl]
                wbuf[t, sl] = x
                acc = acc + x
                acc2 = acc2 + x * x
            lanes = jnp.arange(16, dtype=jnp.int32)
            for k in (8, 4, 2, 1):
                perm = lanes ^ k
                acc = acc + _lane_perm(acc, perm)
                acc2 = acc2 + _lane_perm(acc2, perm)
            mv = acc * (1.0 / HID)
            v = acc2 * (1.0 / HID) - mv * mv + EPS
            vi = lax.bitcast_convert_type(v, jnp.int32)
            y = lax.bitcast_convert_type(
                jnp.int32(0x5F3759DF) - (vi >> 1), jnp.float32)
            y = y * (1.5 - 0.5 * v * y * y)
            y = y * (1.5 - 0.5 * v * y * y)
            y = y * (1.5 - 0.5 * v * y * y)
            for j in range(HC):
                sl = pl.ds(j * 16, 16)
                xn = (wbuf[t, sl] - mv) * y * gv[sl] + bv[sl]
                wbuf[t, sl] = xn
            return carry2

        lax.fori_loop(0, C, tok_body, 0)
        pltpu.sync_copy(wbuf, out.at[pl.ds(base + off, C)])
        return carry

    lax.fori_loop(0, NCH, chunk_body, 0)


def kernel(input_ids, token_type_ids, turn_type_ids, word_table, type_table,
           pos_table, ln_gamma, ln_beta):
    ids = input_ids.reshape(-1)
    tts = token_type_ids.reshape(-1)
    pids = turn_type_ids.reshape(-1)
    mesh = plsc.VectorSubcoreMesh(core_axis_name="c", subcore_axis_name="s")
    f = pl.kernel(
        _body,
        out_type=jax.ShapeDtypeStruct((N, HID), jnp.float32),
        mesh=mesh,
        scratch_types=[
            pltpu.VMEM((TPW,), jnp.int32),
            pltpu.VMEM((TPW,), jnp.int32),
            pltpu.VMEM((TPW,), jnp.int32),
            pltpu.VMEM((C, HID), jnp.float32),
            pltpu.VMEM((C, HID), jnp.float32),
            pltpu.VMEM((C, HID), jnp.float32),
            pltpu.VMEM((HID,), jnp.float32),
            pltpu.VMEM((HID,), jnp.float32),
            pltpu.SemaphoreType.DMA,
        ],
    )
    out = f(ids, tts, pids, word_table, type_table, pos_table, ln_gamma, ln_beta)
    return out.reshape(B, S, HID)
